# Initial kernel scaffold; baseline (speedup 1.0000x reference)
import jax
import jax.numpy as jnp
from jax.experimental import pallas as pl


def kernel(x, edge_attr, edge_index, batch, W0, b0, Wq, bq, Wk, bk, Wv, bv, We, be, Wskip, bskip, W1, b1, W2, b2, W3, b3):
    raise NotImplementedError("write your pallas kernel here")


# speedup vs baseline: 2.0942x; 2.0942x over previous
import functools

import jax
import jax.numpy as jnp
from jax.experimental import pallas as pl
from jax.experimental.pallas import tpu as pltpu

DIM = 64
WVP = 72
NB = 400
EB = 1024
NG = 128


def _leaky(x):
    return jnp.where(x >= 0, x, 0.01 * x)


def _mm_body(x_ref, w_ref, b_ref, o_ref, *, act):
    acc = jnp.dot(x_ref[...], w_ref[...], preferred_element_type=jnp.float32)
    acc = acc + b_ref[...]
    o_ref[...] = _leaky(acc) if act else acc


def _node_matmul(x, w, b, act):
    n, f = x.shape
    fo = w.shape[1]
    grid = n // NB
    return pl.pallas_call(
        functools.partial(_mm_body, act=act),
        grid=(grid,),
        in_specs=[
            pl.BlockSpec((NB, f), lambda i: (i, 0)),
            pl.BlockSpec((f, fo), lambda i: (0, 0)),
            pl.BlockSpec((1, fo), lambda i: (0, 0)),
        ],
        out_specs=pl.BlockSpec((NB, fo), lambda i: (i, 0)),
        out_shape=jax.ShapeDtypeStruct((n, fo), jnp.float32),
    )(x, w, b.reshape(1, fo))


def _edge_body(kvs_ref, qd_ref, ea_ref, we_ref, be_ref, o_ref, *, e_real):
    pid = pl.program_id(0)
    kvs = kvs_ref[...]
    qd = qd_ref[...]
    e = jnp.dot(ea_ref[...], we_ref[...], preferred_element_type=jnp.float32)
    e = e + be_ref[...]
    ks = kvs[:, :DIM] + e
    vs = kvs[:, DIM:] + e
    l = jnp.sum(qd * ks, axis=1, keepdims=True) * (1.0 / 8.0)
    gid = pid * EB + jax.lax.broadcasted_iota(jnp.int32, (EB, 1), 0)
    p = jnp.where(gid < e_real, jnp.exp(l), 0.0)
    o_ref[:, :DIM] = p * vs
    o_ref[:, DIM:DIM + 1] = p
    o_ref[:, DIM + 1:] = jnp.zeros((EB, WVP - DIM - 1), jnp.float32)


def _edge_kernel(kvs, qd, ea, we, be, e_real):
    epad = kvs.shape[0]
    grid = epad // EB
    return pl.pallas_call(
        functools.partial(_edge_body, e_real=e_real),
        grid=(grid,),
        in_specs=[
            pl.BlockSpec((EB, 2 * DIM), lambda i: (i, 0)),
            pl.BlockSpec((EB, DIM), lambda i: (i, 0)),
            pl.BlockSpec((EB, 4), lambda i: (i, 0)),
            pl.BlockSpec((4, DIM), lambda i: (0, 0)),
            pl.BlockSpec((1, DIM), lambda i: (0, 0)),
        ],
        out_specs=pl.BlockSpec((EB, WVP), lambda i: (i, 0)),
        out_shape=jax.ShapeDtypeStruct((epad, WVP), jnp.float32),
    )(kvs, qd, ea, we, be.reshape(1, DIM))


def _update_body(a_ref, out_ref, w_ref, b_ref, o_ref):
    a = a_ref[...]
    skip = jnp.dot(out_ref[...], w_ref[...], preferred_element_type=jnp.float32)
    agg = a[:, :DIM] / (a[:, DIM:DIM + 1] + 1e-16)
    o_ref[...] = _leaky(agg + skip + b_ref[...])


def _update_kernel(a, out, w, b):
    n = out.shape[0]
    return pl.pallas_call(
        _update_body,
        grid=(n // NB,),
        in_specs=[
            pl.BlockSpec((NB, WVP), lambda i: (i, 0)),
            pl.BlockSpec((NB, DIM), lambda i: (i, 0)),
            pl.BlockSpec((DIM, DIM), lambda i: (0, 0)),
            pl.BlockSpec((1, DIM), lambda i: (0, 0)),
        ],
        out_specs=pl.BlockSpec((NB, DIM), lambda i: (i, 0)),
        out_shape=jax.ShapeDtypeStruct((n, DIM), jnp.float32),
    )(a, out, w, b.reshape(1, DIM))


def _pool_body(out_ref, batch_ref, w1_ref, b1_ref, w2_ref, b2_ref, w3_ref,
               b3_ref, o_ref, acc_ref, cnt_ref):
    pid = pl.program_id(0)
    last = pl.num_programs(0) - 1

    @pl.when(pid == 0)
    def _init():
        acc_ref[...] = jnp.zeros_like(acc_ref)
        cnt_ref[...] = jnp.zeros_like(cnt_ref)

    b = batch_ref[0, :, :]
    seg = jax.lax.broadcasted_iota(jnp.int32, (1, NG), 1)
    s = (b == seg).astype(jnp.float32)
    acc_ref[...] += jax.lax.dot_general(
        s, out_ref[...], (((0,), (0,)), ((), ())),
        preferred_element_type=jnp.float32)
    cnt_ref[...] += jnp.sum(s, axis=0, keepdims=True)

    @pl.when(pid == last)
    def _head():
        gm = acc_ref[...] / jnp.maximum(cnt_ref[...], 1.0).reshape(NG, 1)
        h = _leaky(jnp.dot(gm, w1_ref[...], preferred_element_type=jnp.float32)
                   + b1_ref[...])
        h = _leaky(jnp.dot(h, w2_ref[...], preferred_element_type=jnp.float32)
                   + b2_ref[...])
        o_ref[...] = (jnp.dot(h, w3_ref[...], preferred_element_type=jnp.float32)
                      + b3_ref[...])


def _pool_head(out, batch_r, w1, b1, w2, b2, w3, b3):
    n = out.shape[0]
    grid = n // NB
    return pl.pallas_call(
        _pool_body,
        grid=(grid,),
        in_specs=[
            pl.BlockSpec((NB, DIM), lambda i: (i, 0)),
            pl.BlockSpec((1, NB, 1), lambda i: (i, 0, 0)),
            pl.BlockSpec((DIM, DIM), lambda i: (0, 0)),
            pl.BlockSpec((1, DIM), lambda i: (0, 0)),
            pl.BlockSpec((DIM, DIM), lambda i: (0, 0)),
            pl.BlockSpec((1, DIM), lambda i: (0, 0)),
            pl.BlockSpec((DIM, 4), lambda i: (0, 0)),
            pl.BlockSpec((1, 4), lambda i: (0, 0)),
        ],
        out_specs=pl.BlockSpec((NG, 4), lambda i: (0, 0)),
        out_shape=jax.ShapeDtypeStruct((NG, 4), jnp.float32),
        scratch_shapes=[pltpu.VMEM((NG, DIM), jnp.float32),
                        pltpu.VMEM((1, NG), jnp.float32)],
    )(out, batch_r, w1, b1.reshape(1, DIM), w2, b2.reshape(1, DIM),
      w3, b3.reshape(1, 4))


def kernel(x, edge_attr, edge_index, batch, W0, b0, Wq, bq, Wk, bk, Wv, bv,
           We, be, Wskip, bskip, W1, b1, W2, b2, W3, b3):
    n, f_in = x.shape
    e_real = edge_attr.shape[0]
    epad = ((e_real + EB - 1) // EB) * EB
    npad = ((n + NB - 1) // NB) * NB

    src = edge_index[0].astype(jnp.int32)
    dst = edge_index[1].astype(jnp.int32)
    src = jnp.pad(src, (0, epad - e_real))
    dst = jnp.pad(dst, (0, epad - e_real))
    ea = jnp.pad(edge_attr, ((0, epad - e_real), (0, 0)))
    xp = jnp.pad(x, ((0, npad - n), (0, 0)))
    batch_r = jnp.pad(batch.astype(jnp.int32), (0, npad - n),
                      constant_values=NG).reshape(npad // NB, NB, 1)

    wqkv = jnp.concatenate([Wq, Wk, Wv], axis=1)
    bqkv = jnp.concatenate([bq, bk, bv], axis=0)

    out = _node_matmul(xp, W0, b0, act=True)

    for _ in range(12):
        qkv = _node_matmul(out, wqkv, bqkv, act=False)
        q = qkv[:, :DIM]
        kv = qkv[:, DIM:]
        kvs = jnp.take(kv, src, axis=0)
        qd = jnp.take(q, dst, axis=0)
        wvp = _edge_kernel(kvs, qd, ea, We, be, e_real)
        a = jax.ops.segment_sum(wvp, dst, num_segments=npad)
        out = _update_kernel(a, out, Wskip, bskip)

    return _pool_head(out, batch_r, W1, b1, W2, b2, W3, b3)
